# baseline (speedup 1.0000x reference)
import jax
import jax.numpy as jnp
from jax.experimental import pallas as pl
from jax.experimental.pallas import tpu as pltpu

_BH = 176


def _gcn_block_kernel(a1_ref, a2_ref, b_ref, o_ref, b_bf):
    @pl.when(pl.program_id(0) == 0)
    def _():
        b_bf[...] = b_ref[...].astype(jnp.bfloat16)

    b = b_bf[...]
    acc1 = jnp.dot(a1_ref[...].astype(jnp.bfloat16), b,
                   preferred_element_type=jnp.float32)
    o_ref[:_BH, :] = jnp.where(acc1 >= 0, acc1, 0.5 * acc1)
    acc2 = jnp.dot(a2_ref[...].astype(jnp.bfloat16), b,
                   preferred_element_type=jnp.float32)
    o_ref[_BH:, :] = jnp.where(acc2 >= 0, acc2, 0.5 * acc2)


def kernel(adj, embeds):
    n, kdim = adj.shape
    d = embeds.shape[1]
    bm = 2 * _BH
    hmax = (n - 1) // _BH
    return pl.pallas_call(
        _gcn_block_kernel,
        grid=(pl.cdiv(n, bm),),
        in_specs=[
            pl.BlockSpec((_BH, kdim), lambda m: (jnp.minimum(2 * m, hmax), 0)),
            pl.BlockSpec((_BH, kdim),
                         lambda m: (jnp.minimum(2 * m + 1, hmax), 0)),
            pl.BlockSpec((kdim, d), lambda m: (0, 0)),
        ],
        out_specs=pl.BlockSpec((bm, d), lambda m: (m, 0)),
        out_shape=jax.ShapeDtypeStruct((n, d), jnp.float32),
        scratch_shapes=[pltpu.VMEM((kdim, d), jnp.bfloat16)],
    )(adj, adj, embeds)
